# baseline (speedup 1.0000x reference)
import functools

import jax
import jax.numpy as jnp
from jax import lax
from jax.experimental import pallas as pl
from jax.experimental.pallas import tpu as pltpu


def _pool_mask_kernel(ns, s_size, x_ref, w1_ref, b1_ref, w2_ref, b2_ref,
                      g_ref, scal_ref, mask_ref, acc_ref):
    b = pl.program_id(0)
    s = pl.program_id(1)
    nb = pl.num_programs(0)

    part = jnp.sum(x_ref[0].astype(jnp.float32), axis=1)

    @pl.when(s == 0)
    def _init():
        acc_ref[b, :] = part

    @pl.when(s != 0)
    def _acc():
        acc_ref[b, :] = acc_ref[b, :] + part

    @pl.when(jnp.logical_and(b == nb - 1, s == ns - 1))
    def _gate():
        a1 = scal_ref[0]
        a2 = scal_ref[1]
        tf = scal_ref[2]
        pooled = acc_ref[...] / jnp.float32(s_size)
        h = lax.dot_general(pooled, w1_ref[...], (((1,), (1,)), ((), ())),
                            preferred_element_type=jnp.float32)
        h = h + b1_ref[...][None, :]
        h = jnp.where(h >= 0, h, a1 * h)
        h = lax.dot_general(h, w2_ref[...], (((1,), (1,)), ((), ())),
                            preferred_element_type=jnp.float32)
        h = h + b2_ref[...][None, :]
        h = jnp.where(h >= 0, h, a2 * h)
        sft = jax.nn.softmax(h, axis=1)
        mask = jax.nn.softmax(sft + g_ref[...], axis=1)
        idx = jnp.argmax(mask, axis=1)
        iota = lax.broadcasted_iota(jnp.int32, mask.shape, 1)
        hard = jnp.where(iota == idx[:, None], jnp.float32(1), jnp.float32(0))
        mask_ref[...] = jnp.where(tf == 1, hard, mask)


def _wsum_kernel(x_ref, mask_ref, o_ref):
    b = pl.program_id(0)
    xb = x_ref[0]
    acc = mask_ref[b, 0] * xb[0:48, :].astype(jnp.float32)
    for o in range(1, 8):
        acc = acc + mask_ref[b, o] * xb[48 * o:48 * (o + 1), :].astype(jnp.float32)
    o_ref[0] = acc


def kernel(x, W1, b1, a1, W2, b2, a2, test_flag):
    B, C, H, Wd = x.shape
    O = W2.shape[0]
    S = H * Wd
    WP = 256
    SP = H * WP

    xpb = jnp.pad(x, ((0, 0), (0, 0), (0, 0), (0, WP - Wd))).astype(jnp.bfloat16)
    xf = xpb.reshape(B, C, SP)

    u = jax.random.uniform(jax.random.key(42), (B, O),
                           minval=1e-6, maxval=1.0 - 1e-6)
    g = -jnp.log(-jnp.log(u))

    scal = jnp.stack([jnp.float32(a1), jnp.float32(a2),
                      jnp.asarray(test_flag, jnp.float32)])

    NS = 8
    SB = SP // NS

    mask = pl.pallas_call(
        functools.partial(_pool_mask_kernel, NS, S),
        grid=(B, NS),
        in_specs=[
            pl.BlockSpec((1, C, SB), lambda b, s: (b, 0, s)),
            pl.BlockSpec((C, C), lambda b, s: (0, 0)),
            pl.BlockSpec((C,), lambda b, s: (0,)),
            pl.BlockSpec((O, C), lambda b, s: (0, 0)),
            pl.BlockSpec((O,), lambda b, s: (0,)),
            pl.BlockSpec((B, O), lambda b, s: (0, 0)),
            pl.BlockSpec(memory_space=pltpu.SMEM),
        ],
        out_specs=pl.BlockSpec((B, O), lambda b, s: (0, 0)),
        out_shape=jax.ShapeDtypeStruct((B, O), jnp.float32),
        scratch_shapes=[pltpu.VMEM((B, C), jnp.float32)],
        compiler_params=pltpu.CompilerParams(
            dimension_semantics=("arbitrary", "arbitrary")),
    )(xf, W1, b1, W2, b2, g, scal)

    outp = pl.pallas_call(
        _wsum_kernel,
        grid=(B, NS),
        in_specs=[
            pl.BlockSpec((1, C, SB), lambda b, s: (b, 0, s)),
            pl.BlockSpec(memory_space=pltpu.SMEM),
        ],
        out_specs=pl.BlockSpec((1, C // O, SB), lambda b, s: (b, 0, s)),
        out_shape=jax.ShapeDtypeStruct((B, C // O, SP), jnp.float32),
        compiler_params=pltpu.CompilerParams(
            dimension_semantics=("arbitrary", "arbitrary")),
    )(xf, mask)

    out = outp.reshape(B, C // O, H, WP)[:, :, :, :Wd]
    return out, mask.reshape(B, O, 1, 1, 1)


# speedup vs baseline: 1.1010x; 1.1010x over previous
import functools

import jax
import jax.numpy as jnp
from jax import lax
from jax.experimental import pallas as pl
from jax.experimental.pallas import tpu as pltpu


def _pool_mask_kernel(ns, s_size, x_ref, w1_ref, b1_ref, w2_ref, b2_ref,
                      g_ref, scal_ref, mask_ref, acc_ref):
    b = pl.program_id(0)
    s = pl.program_id(1)
    nb = pl.num_programs(0)

    part = jnp.sum(x_ref[0].astype(jnp.float32), axis=(1, 2))

    @pl.when(s == 0)
    def _init():
        acc_ref[b, :] = part

    @pl.when(s != 0)
    def _acc():
        acc_ref[b, :] = acc_ref[b, :] + part

    @pl.when(jnp.logical_and(b == nb - 1, s == ns - 1))
    def _gate():
        a1 = scal_ref[0]
        a2 = scal_ref[1]
        tf = scal_ref[2]
        pooled = acc_ref[...] / jnp.float32(s_size)
        h = lax.dot_general(pooled, w1_ref[...], (((1,), (1,)), ((), ())),
                            preferred_element_type=jnp.float32)
        h = h + b1_ref[...][None, :]
        h = jnp.where(h >= 0, h, a1 * h)
        h = lax.dot_general(h, w2_ref[...], (((1,), (1,)), ((), ())),
                            preferred_element_type=jnp.float32)
        h = h + b2_ref[...][None, :]
        h = jnp.where(h >= 0, h, a2 * h)
        sft = jax.nn.softmax(h, axis=1)
        mask = jax.nn.softmax(sft + g_ref[...], axis=1)
        idx = jnp.argmax(mask, axis=1)
        iota = lax.broadcasted_iota(jnp.int32, mask.shape, 1)
        hard = jnp.where(iota == idx[:, None], jnp.float32(1), jnp.float32(0))
        mask_ref[...] = jnp.where(tf == 1, hard, mask)


def _wsum_kernel(x_ref, mask_ref, o_ref):
    b = pl.program_id(0)
    xb = x_ref[0]
    acc = mask_ref[b, 0] * xb[0:48].astype(jnp.float32)
    for o in range(1, 8):
        acc = acc + mask_ref[b, o] * xb[48 * o:48 * (o + 1)].astype(jnp.float32)
    o_ref[0] = acc


def kernel(x, W1, b1, a1, W2, b2, a2, test_flag):
    B, C, H, Wd = x.shape
    O = W2.shape[0]
    S = H * Wd
    WP = 256

    xpb = jnp.pad(x, ((0, 0), (0, 0), (0, 0), (0, WP - Wd))).astype(jnp.bfloat16)

    u = jax.random.uniform(jax.random.key(42), (B, O),
                           minval=1e-6, maxval=1.0 - 1e-6)
    g = -jnp.log(-jnp.log(u))

    scal = jnp.stack([jnp.float32(a1), jnp.float32(a2),
                      jnp.asarray(test_flag, jnp.float32)])

    NS = 7
    HB = H // NS

    mask = pl.pallas_call(
        functools.partial(_pool_mask_kernel, NS, S),
        grid=(B, NS),
        in_specs=[
            pl.BlockSpec((1, C, HB, WP), lambda b, s: (b, 0, s, 0)),
            pl.BlockSpec((C, C), lambda b, s: (0, 0)),
            pl.BlockSpec((C,), lambda b, s: (0,)),
            pl.BlockSpec((O, C), lambda b, s: (0, 0)),
            pl.BlockSpec((O,), lambda b, s: (0,)),
            pl.BlockSpec((B, O), lambda b, s: (0, 0)),
            pl.BlockSpec(memory_space=pltpu.SMEM),
        ],
        out_specs=pl.BlockSpec((B, O), lambda b, s: (0, 0)),
        out_shape=jax.ShapeDtypeStruct((B, O), jnp.float32),
        scratch_shapes=[pltpu.VMEM((B, C), jnp.float32)],
        compiler_params=pltpu.CompilerParams(
            dimension_semantics=("arbitrary", "arbitrary")),
    )(xpb, W1, b1, W2, b2, g, scal)

    outp = pl.pallas_call(
        _wsum_kernel,
        grid=(B, NS),
        in_specs=[
            pl.BlockSpec((1, C, HB, WP), lambda b, s: (b, 0, s, 0)),
            pl.BlockSpec(memory_space=pltpu.SMEM),
        ],
        out_specs=pl.BlockSpec((1, C // O, HB, WP), lambda b, s: (b, 0, s, 0)),
        out_shape=jax.ShapeDtypeStruct((B, C // O, H, WP), jnp.float32),
        compiler_params=pltpu.CompilerParams(
            dimension_semantics=("arbitrary", "arbitrary")),
    )(xpb, mask)

    return outp[:, :, :, :Wd], mask.reshape(B, O, 1, 1, 1)


# speedup vs baseline: 2.6607x; 2.4165x over previous
import functools

import jax
import jax.numpy as jnp
from jax import lax
from jax.experimental import pallas as pl
from jax.experimental.pallas import tpu as pltpu


def _pool_mask_kernel(ns, s_size, x_ref, w1_ref, b1_ref, w2_ref, b2_ref,
                      g_ref, scal_ref, mask_ref, acc_ref):
    b = pl.program_id(0)
    s = pl.program_id(1)
    nb = pl.num_programs(0)

    part = jnp.sum(x_ref[0], axis=0)

    @pl.when(s == 0)
    def _init():
        acc_ref[b, :] = part

    @pl.when(s != 0)
    def _acc():
        acc_ref[b, :] = acc_ref[b, :] + part

    @pl.when(jnp.logical_and(b == nb - 1, s == ns - 1))
    def _gate():
        a1 = scal_ref[0]
        a2 = scal_ref[1]
        tf = scal_ref[2]
        pooled = acc_ref[...] / jnp.float32(s_size)
        h = lax.dot_general(pooled, w1_ref[...], (((1,), (1,)), ((), ())),
                            preferred_element_type=jnp.float32)
        h = h + b1_ref[...][None, :]
        h = jnp.where(h >= 0, h, a1 * h)
        h = lax.dot_general(h, w2_ref[...], (((1,), (1,)), ((), ())),
                            preferred_element_type=jnp.float32)
        h = h + b2_ref[...][None, :]
        h = jnp.where(h >= 0, h, a2 * h)
        sft = jax.nn.softmax(h, axis=1)
        mask = jax.nn.softmax(sft + g_ref[...], axis=1)
        idx = jnp.argmax(mask, axis=1)
        iota = lax.broadcasted_iota(jnp.int32, mask.shape, 1)
        hard = jnp.where(iota == idx[:, None], jnp.float32(1), jnp.float32(0))
        mask_ref[...] = jnp.where(tf == 1, hard, mask)


def _wsum_kernel(x_ref, mt_ref, o_ref):
    xb = x_ref[0]
    mt = mt_ref[0]
    o_ref[0] = lax.dot_general(mt, xb, (((1,), (1,)), ((), ())),
                               preferred_element_type=jnp.float32)


def kernel(x, W1, b1, a1, W2, b2, a2, test_flag):
    B, C, H, Wd = x.shape
    O = W2.shape[0]
    S = H * Wd
    CB = C // O
    xt = jnp.transpose(x, (0, 2, 3, 1)).reshape(B, S, C)

    u = jax.random.uniform(jax.random.key(42), (B, O),
                           minval=1e-6, maxval=1.0 - 1e-6)
    g = -jnp.log(-jnp.log(u))

    scal = jnp.stack([jnp.float32(a1), jnp.float32(a2),
                      jnp.asarray(test_flag, jnp.float32)])

    NS = 8
    SB = S // NS

    mask = pl.pallas_call(
        functools.partial(_pool_mask_kernel, NS, S),
        grid=(B, NS),
        in_specs=[
            pl.BlockSpec((1, SB, C), lambda b, s: (b, s, 0)),
            pl.BlockSpec((C, C), lambda b, s: (0, 0)),
            pl.BlockSpec((C,), lambda b, s: (0,)),
            pl.BlockSpec((O, C), lambda b, s: (0, 0)),
            pl.BlockSpec((O,), lambda b, s: (0,)),
            pl.BlockSpec((B, O), lambda b, s: (0, 0)),
            pl.BlockSpec(memory_space=pltpu.SMEM),
        ],
        out_specs=pl.BlockSpec((B, O), lambda b, s: (0, 0)),
        out_shape=jax.ShapeDtypeStruct((B, O), jnp.float32),
        scratch_shapes=[pltpu.VMEM((B, C), jnp.float32)],
        compiler_params=pltpu.CompilerParams(
            dimension_semantics=("arbitrary", "arbitrary")),
    )(xt, W1, b1, W2, b2, g, scal)

    c_iota = jnp.arange(C, dtype=jnp.int32)
    sel = (jnp.arange(CB, dtype=jnp.int32)[:, None] == (c_iota % CB)[None, :])
    m_per_c = jnp.repeat(mask, CB, axis=1)
    MT = jnp.where(sel[None], m_per_c[:, None, :], jnp.float32(0))

    outp = pl.pallas_call(
        _wsum_kernel,
        grid=(B, NS),
        in_specs=[
            pl.BlockSpec((1, SB, C), lambda b, s: (b, s, 0)),
            pl.BlockSpec((1, CB, C), lambda b, s: (b, 0, 0)),
        ],
        out_specs=pl.BlockSpec((1, CB, SB), lambda b, s: (b, 0, s)),
        out_shape=jax.ShapeDtypeStruct((B, CB, S), jnp.float32),
        compiler_params=pltpu.CompilerParams(
            dimension_semantics=("arbitrary", "arbitrary")),
    )(xt, MT)

    return outp.reshape(B, CB, H, Wd), mask.reshape(B, O, 1, 1, 1)
